# baseline (speedup 1.0000x reference)
import functools

import jax
import jax.numpy as jnp
from jax import lax
from jax.experimental import pallas as pl
from jax.experimental.pallas import tpu as pltpu
from jax.experimental.pallas import tpu_sc as plsc

BATCH = 16384
EMB = 64
HID = 128

NC = 2
NS = 16
NW = NC * NS
BPW = BATCH // NW
K = 1
NB8 = 8
LG = 128


def _sc_gather(user, item, ut_t, it_t):
    mesh = plsc.VectorSubcoreMesh(core_axis_name="c", subcore_axis_name="s")

    @functools.partial(
        pl.kernel,
        mesh=mesh,
        compiler_params=pltpu.CompilerParams(needs_layout_passes=False),
        out_type=[
            jax.ShapeDtypeStruct((BATCH, EMB), jnp.float32),
            jax.ShapeDtypeStruct((BATCH, EMB), jnp.float32),
        ],
        scratch_types=[
            pltpu.VMEM((BPW,), jnp.int32),
            pltpu.VMEM((NB8, EMB, LG), jnp.float32),
            pltpu.VMEM((BPW // 2, EMB), jnp.float32),
        ] + [pltpu.SemaphoreType.DMA] * NB8,
    )
    def gather_kernel(user_hbm, item_hbm, ut_hbm, it_hbm, uo_hbm, io_hbm,
                      idx_v, banks8, rows_v, *sems8):
        wid = lax.axis_index("s") * NC + lax.axis_index("c")
        base = wid * BPW
        iota16 = lax.iota(jnp.int32, 16)
        NS_SWEEPS = BPW // 16

        def do_table(idx_hbm, tbl_hbm, out_hbm):
            pltpu.sync_copy(idx_hbm.at[pl.ds(base, BPW)], idx_v)

            def fire(rbvec, q, b):
                rb = pl.multiple_of(rbvec[q], LG)
                for h in range(4):
                    pltpu.async_copy(
                        tbl_hbm.at[pl.ds(16 * h, 16), pl.ds(rb, LG)],
                        banks8.at[b].at[pl.ds(16 * h, 16)], sems8[b])

            def drain(b):
                pltpu.make_async_copy(
                    tbl_hbm.at[:, pl.ds(0, LG)], banks8.at[b],
                    sems8[b]).wait()

            def extract(lvec, sweep, q, b):
                j = (sweep % (NS_SWEEPS // 2)) * 16 + q
                lane = jnp.broadcast_to(lvec[q], (16,))
                for m in range(EMB // 16):
                    v = plsc.load_gather(
                        banks8.at[b], [iota16 + 16 * m, lane])
                    rows_v[j, pl.ds(16 * m, 16)] = v

            def sweep_vecs(i):
                rvec = idx_v[pl.ds(i * 16, 16)]
                return (rvec >> 7) << 7, rvec & 127

            rb0, _ = sweep_vecs(0)
            for q in range(8):
                fire(rb0, q, q)

            def body(i, carry):
                @pl.when(i == NS_SWEEPS // 2)
                def _():
                    pltpu.sync_copy(rows_v, out_hbm.at[pl.ds(base, BPW // 2)])

                rbvec, lvec = sweep_vecs(i)
                inext = lax.rem(i + 1, NS_SWEEPS)
                rbnext, _ = sweep_vecs(inext)
                for q in range(8):
                    drain(q)
                    extract(lvec, i, q, q)
                    fire(rbvec, 8 + q, q)
                for q in range(8):
                    drain(q)
                    extract(lvec, i, 8 + q, q)
                    fire(rbnext, q, q)
                return carry

            lax.fori_loop(0, NS_SWEEPS, body, 0)
            for q in range(8):
                drain(q)
            pltpu.sync_copy(rows_v, out_hbm.at[pl.ds(base + BPW // 2, BPW // 2)])

        do_table(user_hbm, ut_hbm, uo_hbm)
        do_table(item_hbm, it_hbm, io_hbm)

    return gather_kernel(user, item, ut_t, it_t)


BLK = 2048


def _mlp_body(u_ref, i_ref, w1u_ref, w1i_ref, b1_ref, w2_ref, b2_ref, o_ref):
    xu = lax.dot_general(u_ref[...], w1u_ref[...], (((1,), (0,)), ((), ())),
                         preferred_element_type=jnp.float32)
    xi = lax.dot_general(i_ref[...], w1i_ref[...], (((1,), (0,)), ((), ())),
                         preferred_element_type=jnp.float32)
    h = jnp.maximum(xu + xi + b1_ref[...], 0.0)
    y = lax.dot_general(h, w2_ref[...], (((1,), (0,)), ((), ())),
                        preferred_element_type=jnp.float32)
    o_ref[...] = 4.0 * jax.nn.sigmoid(y[:, 0:1] + b2_ref[0]) + 1.0


def _tc_mlp(uemb, iemb, w1u, w1i, b1, w2, b2):
    grid = (BATCH // BLK,)
    return pl.pallas_call(
        _mlp_body,
        grid=grid,
        in_specs=[
            pl.BlockSpec((BLK, EMB), lambda b: (b, 0)),
            pl.BlockSpec((BLK, EMB), lambda b: (b, 0)),
            pl.BlockSpec((EMB, HID), lambda b: (0, 0)),
            pl.BlockSpec((EMB, HID), lambda b: (0, 0)),
            pl.BlockSpec((1, HID), lambda b: (0, 0)),
            pl.BlockSpec((HID, 128), lambda b: (0, 0)),
            pl.BlockSpec(memory_space=pltpu.SMEM),
        ],
        out_specs=pl.BlockSpec((BLK, 1), lambda b: (b, 0)),
        out_shape=jax.ShapeDtypeStruct((BATCH, 1), jnp.float32),
    )(uemb, iemb, w1u, w1i, b1, w2, b2)


@jax.jit
def _run(user, item, user_table, item_table, W1, b1, W2, b2):
    uemb, iemb = _sc_gather(user.astype(jnp.int32), item.astype(jnp.int32),
                            user_table.T, item_table.T)
    w1u = W1[:, :EMB].T
    w1i = W1[:, EMB:].T
    w2pad = jnp.zeros((HID, 128), jnp.float32).at[:, 0].set(W2[0])
    out = _tc_mlp(uemb, iemb, w1u, w1i, b1.reshape(1, HID), w2pad, b2)
    return out.reshape(-1)


def kernel(user, item, user_table, item_table, W1, b1, W2, b2):
    return _run(user, item, user_table, item_table, W1, b1, W2, b2)
